# baseline (speedup 1.0000x reference)
import functools

import jax
import jax.numpy as jnp
from jax import lax
from jax.experimental import pallas as pl
from jax.experimental.pallas import tpu as pltpu
from jax.experimental.pallas import tpu_sc as plsc

NUM_CORES = 2
NUM_SUBCORES = 16
LANES = 16
NUM_WORKERS = NUM_CORES * NUM_SUBCORES
CHUNK = 128
UNROLL = 8


def _dot_head_kernel(d, span, n, table_hbm, src_idx_hbm, tgt_idx_hbm, out_hbm,
                     table_sh, sidx_v, tidx_v, src_v, tgt_v, part_v, out_v,
                     tsidx_v, ttidx_v, tsrc_v, ttgt_v, tout_v,
                     isem0, isem1, glo0, glo1, ghi0, ghi1, osem0, osem1):
    wid = lax.axis_index("s") * NUM_CORES + lax.axis_index("c")
    sid = lax.axis_index("s")
    base = wid * span
    rows_per_sub = n // NUM_SUBCORES
    pltpu.sync_copy(table_hbm.at[pl.ds(sid * rows_per_sub, rows_per_sub)],
                    table_sh.at[pl.ds(sid * rows_per_sub, rows_per_sub)])
    plsc.subcore_barrier()
    n_full = span // CHUNK
    tail = span - n_full * CHUNK
    inv_d = 1.0 / d
    words = d // 2
    lane = lax.iota(jnp.int32, LANES)
    isem = (isem0, isem1)
    glo = (glo0, glo1)
    ghi = (ghi0, ghi1)
    osem = (osem0, osem1)
    H = CHUNK // 2

    def issue_idx(jv, p):
        off = base + jv * CHUNK
        pltpu.async_copy(src_idx_hbm.at[pl.ds(off, CHUNK)], sidx_v.at[p],
                         isem[p])
        pltpu.async_copy(tgt_idx_hbm.at[pl.ds(off, CHUNK)], tidx_v.at[p],
                         isem[p])

    def wait_idx(p):
        pltpu.make_async_copy(src_idx_hbm.at[pl.ds(base, CHUNK)],
                              sidx_v.at[p], isem[p]).wait()
        pltpu.make_async_copy(tgt_idx_hbm.at[pl.ds(base, CHUNK)],
                              tidx_v.at[p], isem[p]).wait()

    def issue_gathers(p):
        pltpu.async_copy(table_sh.at[sidx_v.at[p, pl.ds(0, H)]],
                         src_v.at[p, pl.ds(0, H)], glo[p])
        pltpu.async_copy(table_sh.at[tidx_v.at[p, pl.ds(0, H)]],
                         tgt_v.at[p, pl.ds(0, H)], glo[p])
        pltpu.async_copy(table_sh.at[sidx_v.at[p, pl.ds(H, H)]],
                         src_v.at[p, pl.ds(H, H)], ghi[p])
        pltpu.async_copy(table_sh.at[tidx_v.at[p, pl.ds(H, H)]],
                         tgt_v.at[p, pl.ds(H, H)], ghi[p])

    def wait_ghalf(p, sems):
        pltpu.make_async_copy(table_sh.at[sidx_v.at[p, pl.ds(0, H)]],
                              src_v.at[p, pl.ds(0, H)], sems[p]).wait()
        pltpu.make_async_copy(table_sh.at[tidx_v.at[p, pl.ds(0, H)]],
                              tgt_v.at[p, pl.ds(0, H)], sems[p]).wait()

    def wait_out(p):
        pltpu.make_async_copy(out_v.at[p], out_hbm.at[pl.ds(base, CHUNK)],
                              osem[p]).wait()

    def dot_rows(size, src_b, tgt_b, out_b):
        @plsc.parallel_loop(0, size, unroll=UNROLL)
        def _edge(e):
            prods = []
            for k in range(words // LANES):
                s_bf = plsc.bitcast(src_b[e, pl.ds(k * LANES, LANES)],
                                    jnp.bfloat16)
                t_bf = plsc.bitcast(tgt_b[e, pl.ds(k * LANES, LANES)],
                                    jnp.bfloat16)
                prods.append(s_bf * t_bf)
            while len(prods) > 1:
                prods = [prods[i] + prods[i + 1]
                         for i in range(0, len(prods), 2)]
            lo, hi = plsc.unpack(prods[0], format=plsc.PackFormat.INTERLEAVED)
            part_v[e, :] = lo + hi

        for g in range(size // LANES):
            row = g * LANES + lane
            tot = plsc.load_gather(part_v, [row, jnp.zeros_like(lane)])
            for c in range(1, LANES):
                tot += plsc.load_gather(part_v, [row, jnp.full_like(lane, c)])
            out_b[pl.ds(g * LANES, LANES)] = tot * inv_d

    def compute(jv, p):
        dot_rows(H, src_v.at[p, pl.ds(0, H)], tgt_v.at[p, pl.ds(0, H)],
                 out_v.at[p, pl.ds(0, H)])
        wait_ghalf(p, ghi)
        dot_rows(H, src_v.at[p, pl.ds(H, H)], tgt_v.at[p, pl.ds(H, H)],
                 out_v.at[p, pl.ds(H, H)])
        off = base + jv * CHUNK
        pltpu.async_copy(out_v.at[p], out_hbm.at[pl.ds(off, CHUNK)], osem[p])

    def stage(jv, p):
        @pl.when(jv + 1 < n_full)
        def _():
            wait_idx(1 - p)
            issue_gathers(1 - p)

        wait_ghalf(p, glo)

        @pl.when(jv + 2 < n_full)
        def _():
            issue_idx(jv + 2, p)

        @pl.when(jv >= 2)
        def _():
            wait_out(p)

        compute(jv, p)

    issue_idx(0, 0)
    wait_idx(0)
    issue_gathers(0)
    issue_idx(1, 1)

    @pl.loop(0, n_full, step=2)
    def _pair(j):
        stage(j, 0)
        stage(j + 1, 1)

    wait_out(0)
    wait_out(1)

    if tail:
        off = base + n_full * CHUNK
        pltpu.sync_copy(src_idx_hbm.at[pl.ds(off, tail)], tsidx_v)
        pltpu.sync_copy(tgt_idx_hbm.at[pl.ds(off, tail)], ttidx_v)
        c0 = pltpu.async_copy(table_sh.at[tsidx_v], tsrc_v, glo0)
        c1 = pltpu.async_copy(table_sh.at[ttidx_v], ttgt_v, glo1)
        c0.wait()
        c1.wait()
        dot_rows(tail, tsrc_v, ttgt_v, tout_v)
        pltpu.sync_copy(tout_v, out_hbm.at[pl.ds(off, tail)])


def kernel(node_embeddings, edge_index):
    n, d = node_embeddings.shape
    b = edge_index.shape[1]
    assert d % (2 * LANES) == 0
    assert b % NUM_WORKERS == 0
    assert n % NUM_SUBCORES == 0
    span = b // NUM_WORKERS
    n_full = span // CHUNK
    tail = span - n_full * CHUNK
    assert n_full % 2 == 0 and n_full >= 4
    assert tail % LANES == 0 and tail % 8 == 0

    table_bf = node_embeddings.astype(jnp.bfloat16)
    table_pk = lax.bitcast_convert_type(
        table_bf.reshape(n, d // 2, 2), jnp.int32)
    edge_index = edge_index.astype(jnp.int32)
    src_idx = edge_index[0]
    tgt_idx = edge_index[1]

    mesh = plsc.VectorSubcoreMesh(core_axis_name="c", subcore_axis_name="s")
    run = pl.kernel(
        functools.partial(_dot_head_kernel, d, span, n),
        out_type=jax.ShapeDtypeStruct((b,), jnp.float32),
        mesh=mesh,
        compiler_params=pltpu.CompilerParams(
            needs_layout_passes=False, use_tc_tiling_on_sc=False),
        scratch_types=[
            pltpu.VMEM_SHARED((n, d // 2), jnp.int32),
            pltpu.VMEM((2, CHUNK), jnp.int32),
            pltpu.VMEM((2, CHUNK), jnp.int32),
            pltpu.VMEM((2, CHUNK, d // 2), jnp.int32),
            pltpu.VMEM((2, CHUNK, d // 2), jnp.int32),
            pltpu.VMEM((CHUNK, LANES), jnp.float32),
            pltpu.VMEM((2, CHUNK), jnp.float32),
            pltpu.VMEM((max(tail, 8),), jnp.int32),
            pltpu.VMEM((max(tail, 8),), jnp.int32),
            pltpu.VMEM((max(tail, 8), d // 2), jnp.int32),
            pltpu.VMEM((max(tail, 8), d // 2), jnp.int32),
            pltpu.VMEM((max(tail, 8),), jnp.float32),
            pltpu.SemaphoreType.DMA,
            pltpu.SemaphoreType.DMA,
            pltpu.SemaphoreType.DMA,
            pltpu.SemaphoreType.DMA,
            pltpu.SemaphoreType.DMA,
            pltpu.SemaphoreType.DMA,
            pltpu.SemaphoreType.DMA,
            pltpu.SemaphoreType.DMA,
        ],
    )
    return run(table_pk, src_idx, tgt_idx)


# speedup vs baseline: 1.0299x; 1.0299x over previous
import functools

import jax
import jax.numpy as jnp
from jax import lax
from jax.experimental import pallas as pl
from jax.experimental.pallas import tpu as pltpu
from jax.experimental.pallas import tpu_sc as plsc

NUM_CORES = 2
NUM_SUBCORES = 16
LANES = 16
NUM_WORKERS = NUM_CORES * NUM_SUBCORES
CHUNK = 128
UNROLL = 8


def _dot_head_kernel(d, span, n, table_hbm, src_idx_hbm, tgt_idx_hbm, out_hbm,
                     table_sh, sidx_v, tidx_v, src_v, tgt_v, part_v, out_v,
                     tsidx_v, ttidx_v, tsrc_v, ttgt_v, tout_v,
                     isem0, isem1, glo0, glo1, ghi0, ghi1, osem0, osem1):
    wid = lax.axis_index("s") * NUM_CORES + lax.axis_index("c")
    sid = lax.axis_index("s")
    base = wid * span
    rows_per_sub = n // NUM_SUBCORES
    pltpu.sync_copy(table_hbm.at[pl.ds(sid * rows_per_sub, rows_per_sub)],
                    table_sh.at[pl.ds(sid * rows_per_sub, rows_per_sub)])
    plsc.subcore_barrier()
    n_full = span // CHUNK
    tail = span - n_full * CHUNK
    inv_d = 1.0 / d
    words = d // 2
    lane = lax.iota(jnp.int32, LANES)
    isem = (isem0, isem1)
    glo = (glo0, glo1)
    ghi = (ghi0, ghi1)
    osem = (osem0, osem1)
    H = CHUNK // 2

    def issue_idx(jv, p):
        off = base + jv * CHUNK
        pltpu.async_copy(src_idx_hbm.at[pl.ds(off, CHUNK)], sidx_v.at[p],
                         isem[p])
        pltpu.async_copy(tgt_idx_hbm.at[pl.ds(off, CHUNK)], tidx_v.at[p],
                         isem[p])

    def wait_idx(p):
        pltpu.make_async_copy(src_idx_hbm.at[pl.ds(base, CHUNK)],
                              sidx_v.at[p], isem[p]).wait()
        pltpu.make_async_copy(tgt_idx_hbm.at[pl.ds(base, CHUNK)],
                              tidx_v.at[p], isem[p]).wait()

    def issue_gathers(p):
        pltpu.async_copy(table_sh.at[sidx_v.at[p, pl.ds(0, H)]],
                         src_v.at[p, pl.ds(0, H)], glo[p])
        pltpu.async_copy(table_sh.at[tidx_v.at[p, pl.ds(0, H)]],
                         tgt_v.at[p, pl.ds(0, H)], glo[p])
        pltpu.async_copy(table_sh.at[sidx_v.at[p, pl.ds(H, H)]],
                         src_v.at[p, pl.ds(H, H)], ghi[p])
        pltpu.async_copy(table_sh.at[tidx_v.at[p, pl.ds(H, H)]],
                         tgt_v.at[p, pl.ds(H, H)], ghi[p])

    def wait_ghalf(p, sems):
        pltpu.make_async_copy(table_sh.at[sidx_v.at[p, pl.ds(0, H)]],
                              src_v.at[p, pl.ds(0, H)], sems[p]).wait()
        pltpu.make_async_copy(table_sh.at[tidx_v.at[p, pl.ds(0, H)]],
                              tgt_v.at[p, pl.ds(0, H)], sems[p]).wait()

    def wait_out(p):
        pltpu.make_async_copy(out_v.at[p], out_hbm.at[pl.ds(base, CHUNK)],
                              osem[p]).wait()

    def dot_rows(size, src_b, tgt_b, out_b):
        @plsc.parallel_loop(0, size, unroll=UNROLL)
        def _edge(e):
            acc = None
            for k in range(words // LANES):
                s_bf = plsc.bitcast(src_b[e, pl.ds(k * LANES, LANES)],
                                    jnp.bfloat16)
                t_bf = plsc.bitcast(tgt_b[e, pl.ds(k * LANES, LANES)],
                                    jnp.bfloat16)
                lo, hi = plsc.unpack(s_bf * t_bf,
                                     format=plsc.PackFormat.INTERLEAVED)
                acc = lo + hi if acc is None else acc + lo + hi
            part_v[e, :] = acc

        for g in range(size // LANES):
            row = g * LANES + lane
            tot = plsc.load_gather(part_v, [row, jnp.zeros_like(lane)])
            for c in range(1, LANES):
                tot += plsc.load_gather(part_v, [row, jnp.full_like(lane, c)])
            out_b[pl.ds(g * LANES, LANES)] = tot * inv_d

    def compute(jv, p):
        dot_rows(H, src_v.at[p, pl.ds(0, H)], tgt_v.at[p, pl.ds(0, H)],
                 out_v.at[p, pl.ds(0, H)])
        wait_ghalf(p, ghi)
        dot_rows(H, src_v.at[p, pl.ds(H, H)], tgt_v.at[p, pl.ds(H, H)],
                 out_v.at[p, pl.ds(H, H)])
        off = base + jv * CHUNK
        pltpu.async_copy(out_v.at[p], out_hbm.at[pl.ds(off, CHUNK)], osem[p])

    def stage(jv, p):
        @pl.when(jv + 1 < n_full)
        def _():
            wait_idx(1 - p)
            issue_gathers(1 - p)

        wait_ghalf(p, glo)

        @pl.when(jv + 2 < n_full)
        def _():
            issue_idx(jv + 2, p)

        @pl.when(jv >= 2)
        def _():
            wait_out(p)

        compute(jv, p)

    issue_idx(0, 0)
    wait_idx(0)
    issue_gathers(0)
    issue_idx(1, 1)

    @pl.loop(0, n_full, step=2)
    def _pair(j):
        stage(j, 0)
        stage(j + 1, 1)

    wait_out(0)
    wait_out(1)

    if tail:
        off = base + n_full * CHUNK
        pltpu.sync_copy(src_idx_hbm.at[pl.ds(off, tail)], tsidx_v)
        pltpu.sync_copy(tgt_idx_hbm.at[pl.ds(off, tail)], ttidx_v)
        c0 = pltpu.async_copy(table_sh.at[tsidx_v], tsrc_v, glo0)
        c1 = pltpu.async_copy(table_sh.at[ttidx_v], ttgt_v, glo1)
        c0.wait()
        c1.wait()
        dot_rows(tail, tsrc_v, ttgt_v, tout_v)
        pltpu.sync_copy(tout_v, out_hbm.at[pl.ds(off, tail)])


def kernel(node_embeddings, edge_index):
    n, d = node_embeddings.shape
    b = edge_index.shape[1]
    assert d % (2 * LANES) == 0
    assert b % NUM_WORKERS == 0
    assert n % NUM_SUBCORES == 0
    span = b // NUM_WORKERS
    n_full = span // CHUNK
    tail = span - n_full * CHUNK
    assert n_full % 2 == 0 and n_full >= 4
    assert tail % LANES == 0 and tail % 8 == 0

    table_bf = node_embeddings.astype(jnp.bfloat16)
    table_pk = lax.bitcast_convert_type(
        table_bf.reshape(n, d // 2, 2), jnp.int32)
    edge_index = edge_index.astype(jnp.int32)
    src_idx = edge_index[0]
    tgt_idx = edge_index[1]

    mesh = plsc.VectorSubcoreMesh(core_axis_name="c", subcore_axis_name="s")
    run = pl.kernel(
        functools.partial(_dot_head_kernel, d, span, n),
        out_type=jax.ShapeDtypeStruct((b,), jnp.float32),
        mesh=mesh,
        compiler_params=pltpu.CompilerParams(
            needs_layout_passes=False, use_tc_tiling_on_sc=False),
        scratch_types=[
            pltpu.VMEM_SHARED((n, d // 2), jnp.int32),
            pltpu.VMEM((2, CHUNK), jnp.int32),
            pltpu.VMEM((2, CHUNK), jnp.int32),
            pltpu.VMEM((2, CHUNK, d // 2), jnp.int32),
            pltpu.VMEM((2, CHUNK, d // 2), jnp.int32),
            pltpu.VMEM((CHUNK, LANES), jnp.float32),
            pltpu.VMEM((2, CHUNK), jnp.float32),
            pltpu.VMEM((max(tail, 8),), jnp.int32),
            pltpu.VMEM((max(tail, 8),), jnp.int32),
            pltpu.VMEM((max(tail, 8), d // 2), jnp.int32),
            pltpu.VMEM((max(tail, 8), d // 2), jnp.int32),
            pltpu.VMEM((max(tail, 8),), jnp.float32),
            pltpu.SemaphoreType.DMA,
            pltpu.SemaphoreType.DMA,
            pltpu.SemaphoreType.DMA,
            pltpu.SemaphoreType.DMA,
            pltpu.SemaphoreType.DMA,
            pltpu.SemaphoreType.DMA,
            pltpu.SemaphoreType.DMA,
            pltpu.SemaphoreType.DMA,
        ],
    )
    return run(table_pk, src_idx, tgt_idx)


# speedup vs baseline: 1.0375x; 1.0073x over previous
import functools

import jax
import jax.numpy as jnp
from jax import lax
from jax.experimental import pallas as pl
from jax.experimental.pallas import tpu as pltpu
from jax.experimental.pallas import tpu_sc as plsc

NUM_CORES = 2
NUM_SUBCORES = 16
LANES = 16
NUM_WORKERS = NUM_CORES * NUM_SUBCORES
CHUNK = 128
UNROLL = 8


def _dot_head_kernel(d, span, n, table_hbm, src_idx_hbm, tgt_idx_hbm, out_hbm,
                     table_sh, sidx_v, tidx_v, src_v, tgt_v, part_v, out_v,
                     tsidx_v, ttidx_v, tsrc_v, ttgt_v, tout_v,
                     isem0, isem1, glo0, glo1, ghi0, ghi1, osem0, osem1):
    wid = lax.axis_index("s") * NUM_CORES + lax.axis_index("c")
    sid = lax.axis_index("s")
    base = wid * span
    rows_per_sub = n // NUM_SUBCORES
    n_full = span // CHUNK
    tail = span - n_full * CHUNK
    inv_d = 1.0 / d
    words = d // 2
    lane = lax.iota(jnp.int32, LANES)
    isem = (isem0, isem1)
    glo = (glo0, glo1)
    ghi = (ghi0, ghi1)
    osem = (osem0, osem1)
    H = CHUNK // 2

    def issue_idx(jv, p):
        off = base + jv * CHUNK
        pltpu.async_copy(src_idx_hbm.at[pl.ds(off, CHUNK)], sidx_v.at[p],
                         isem[p])
        pltpu.async_copy(tgt_idx_hbm.at[pl.ds(off, CHUNK)], tidx_v.at[p],
                         isem[p])

    def wait_idx(p):
        pltpu.make_async_copy(src_idx_hbm.at[pl.ds(base, CHUNK)],
                              sidx_v.at[p], isem[p]).wait()
        pltpu.make_async_copy(tgt_idx_hbm.at[pl.ds(base, CHUNK)],
                              tidx_v.at[p], isem[p]).wait()

    def issue_gathers(p, tbl=None):
        t = table_sh if tbl is None else tbl
        pltpu.async_copy(t.at[sidx_v.at[p, pl.ds(0, H)]],
                         src_v.at[p, pl.ds(0, H)], glo[p])
        pltpu.async_copy(t.at[tidx_v.at[p, pl.ds(0, H)]],
                         tgt_v.at[p, pl.ds(0, H)], glo[p])
        pltpu.async_copy(t.at[sidx_v.at[p, pl.ds(H, H)]],
                         src_v.at[p, pl.ds(H, H)], ghi[p])
        pltpu.async_copy(t.at[tidx_v.at[p, pl.ds(H, H)]],
                         tgt_v.at[p, pl.ds(H, H)], ghi[p])

    def wait_ghalf(p, sems):
        pltpu.make_async_copy(table_sh.at[sidx_v.at[p, pl.ds(0, H)]],
                              src_v.at[p, pl.ds(0, H)], sems[p]).wait()
        pltpu.make_async_copy(table_sh.at[tidx_v.at[p, pl.ds(0, H)]],
                              tgt_v.at[p, pl.ds(0, H)], sems[p]).wait()

    def wait_out(p):
        pltpu.make_async_copy(out_v.at[p], out_hbm.at[pl.ds(base, CHUNK)],
                              osem[p]).wait()

    def dot_rows(size, src_b, tgt_b, out_b):
        @plsc.parallel_loop(0, size, unroll=UNROLL)
        def _edge(e):
            acc = None
            for k in range(words // LANES):
                s_bf = plsc.bitcast(src_b[e, pl.ds(k * LANES, LANES)],
                                    jnp.bfloat16)
                t_bf = plsc.bitcast(tgt_b[e, pl.ds(k * LANES, LANES)],
                                    jnp.bfloat16)
                lo, hi = plsc.unpack(s_bf * t_bf,
                                     format=plsc.PackFormat.INTERLEAVED)
                acc = lo + hi if acc is None else acc + lo + hi
            part_v[e, :] = acc

        for g in range(size // LANES):
            row = g * LANES + lane
            tot = plsc.load_gather(part_v, [row, jnp.zeros_like(lane)])
            for c in range(1, LANES):
                tot += plsc.load_gather(part_v, [row, jnp.full_like(lane, c)])
            out_b[pl.ds(g * LANES, LANES)] = tot * inv_d

    def compute(jv, p):
        dot_rows(H, src_v.at[p, pl.ds(0, H)], tgt_v.at[p, pl.ds(0, H)],
                 out_v.at[p, pl.ds(0, H)])
        wait_ghalf(p, ghi)
        dot_rows(H, src_v.at[p, pl.ds(H, H)], tgt_v.at[p, pl.ds(H, H)],
                 out_v.at[p, pl.ds(H, H)])
        off = base + jv * CHUNK
        pltpu.async_copy(out_v.at[p], out_hbm.at[pl.ds(off, CHUNK)], osem[p])

    def stage(jv, p):
        @pl.when(jv + 1 < n_full)
        def _():
            wait_idx(1 - p)
            issue_gathers(1 - p)

        wait_ghalf(p, glo)

        @pl.when(jv + 2 < n_full)
        def _():
            issue_idx(jv + 2, p)

        @pl.when(jv >= 2)
        def _():
            wait_out(p)

        compute(jv, p)

    issue_idx(0, 0)
    wait_idx(0)
    issue_gathers(0, tbl=table_hbm)
    issue_idx(1, 1)
    pltpu.sync_copy(table_hbm.at[pl.ds(sid * rows_per_sub, rows_per_sub)],
                    table_sh.at[pl.ds(sid * rows_per_sub, rows_per_sub)])
    plsc.subcore_barrier()

    @pl.loop(0, n_full, step=2)
    def _pair(j):
        stage(j, 0)
        stage(j + 1, 1)

    wait_out(0)
    wait_out(1)

    if tail:
        off = base + n_full * CHUNK
        pltpu.sync_copy(src_idx_hbm.at[pl.ds(off, tail)], tsidx_v)
        pltpu.sync_copy(tgt_idx_hbm.at[pl.ds(off, tail)], ttidx_v)
        c0 = pltpu.async_copy(table_sh.at[tsidx_v], tsrc_v, glo0)
        c1 = pltpu.async_copy(table_sh.at[ttidx_v], ttgt_v, glo1)
        c0.wait()
        c1.wait()
        dot_rows(tail, tsrc_v, ttgt_v, tout_v)
        pltpu.sync_copy(tout_v, out_hbm.at[pl.ds(off, tail)])


def kernel(node_embeddings, edge_index):
    n, d = node_embeddings.shape
    b = edge_index.shape[1]
    assert d % (2 * LANES) == 0
    assert b % NUM_WORKERS == 0
    assert n % NUM_SUBCORES == 0
    span = b // NUM_WORKERS
    n_full = span // CHUNK
    tail = span - n_full * CHUNK
    assert n_full % 2 == 0 and n_full >= 4
    assert tail % LANES == 0 and tail % 8 == 0

    table_bf = node_embeddings.astype(jnp.bfloat16)
    table_pk = lax.bitcast_convert_type(
        table_bf.reshape(n, d // 2, 2), jnp.int32)
    edge_index = edge_index.astype(jnp.int32)
    src_idx = edge_index[0]
    tgt_idx = edge_index[1]

    mesh = plsc.VectorSubcoreMesh(core_axis_name="c", subcore_axis_name="s")
    run = pl.kernel(
        functools.partial(_dot_head_kernel, d, span, n),
        out_type=jax.ShapeDtypeStruct((b,), jnp.float32),
        mesh=mesh,
        compiler_params=pltpu.CompilerParams(
            needs_layout_passes=False, use_tc_tiling_on_sc=False),
        scratch_types=[
            pltpu.VMEM_SHARED((n, d // 2), jnp.int32),
            pltpu.VMEM((2, CHUNK), jnp.int32),
            pltpu.VMEM((2, CHUNK), jnp.int32),
            pltpu.VMEM((2, CHUNK, d // 2), jnp.int32),
            pltpu.VMEM((2, CHUNK, d // 2), jnp.int32),
            pltpu.VMEM((CHUNK, LANES), jnp.float32),
            pltpu.VMEM((2, CHUNK), jnp.float32),
            pltpu.VMEM((max(tail, 8),), jnp.int32),
            pltpu.VMEM((max(tail, 8),), jnp.int32),
            pltpu.VMEM((max(tail, 8), d // 2), jnp.int32),
            pltpu.VMEM((max(tail, 8), d // 2), jnp.int32),
            pltpu.VMEM((max(tail, 8),), jnp.float32),
            pltpu.SemaphoreType.DMA,
            pltpu.SemaphoreType.DMA,
            pltpu.SemaphoreType.DMA,
            pltpu.SemaphoreType.DMA,
            pltpu.SemaphoreType.DMA,
            pltpu.SemaphoreType.DMA,
            pltpu.SemaphoreType.DMA,
            pltpu.SemaphoreType.DMA,
        ],
    )
    return run(table_pk, src_idx, tgt_idx)
